# baseline (device time: 144659 ns/iter reference)
import os

import jax
import jax.numpy as jnp
from jax import lax
from jax.experimental import pallas as pl
from jax.experimental.pallas import tpu as pltpu

_ABLATE = os.environ.get("ABLATE", "")

N_DEV = 4
B, S, H, D = 2, 512, 8, 64
SCALE = D ** -0.5


def kernel(Q, K, V):
    def body(q_ref, k_ref, v_ref, out_ref,
             qt, kloc, vloc, k_com, v_com, acc, lsum,
             ksend, krecv, vsend, vrecv):
        mx = lax.axis_index("x")
        my = lax.axis_index("y")
        mz = lax.axis_index("z")

        barrier_sem = pltpu.get_barrier_semaphore()
        for j in range(N_DEV - 1):
            pl.semaphore_signal(
                barrier_sem, inc=1,
                device_id=(mx, my, (mz + 1 + j) % N_DEV),
                device_id_type=pl.DeviceIdType.MESH,
            )
        pl.semaphore_wait(barrier_sem, N_DEV - 1)

        for b in range(B):
            kloc[b] = jnp.transpose(k_ref[b], (1, 2, 0)).astype(jnp.bfloat16)
            vloc[b] = jnp.transpose(v_ref[b], (1, 2, 0)).astype(jnp.bfloat16)

        sends = []
        for j in range(N_DEV - 1 if _ABLATE != "compute" else 0):
            p = (mz + 1 + j) % N_DEV
            r = N_DEV - 2 - j
            rk = pltpu.make_async_remote_copy(
                src_ref=kloc, dst_ref=k_com.at[r],
                send_sem=ksend.at[j], recv_sem=krecv.at[r],
                device_id=(mx, my, p), device_id_type=pl.DeviceIdType.MESH,
            )
            rv = pltpu.make_async_remote_copy(
                src_ref=vloc, dst_ref=v_com.at[r],
                send_sem=vsend.at[j], recv_sem=vrecv.at[r],
                device_id=(mx, my, p), device_id_type=pl.DeviceIdType.MESH,
            )
            rk.start()
            rv.start()
            sends.append((rk, rv))

        for b in range(B):
            qt[b] = (jnp.transpose(q_ref[b], (1, 0, 2)) * SCALE
                     ).astype(jnp.bfloat16)

        def chunk_pass(k_at, v_at, first, last):
            def bh_step(bh, _):
                b = bh // H
                hh = bh % H
                q = qt[b, hh]
                kT = k_at(b, hh)
                vT = v_at(b, hh)
                s = lax.dot_general(
                    q, kT, (((1,), (0,)), ((), ())),
                    preferred_element_type=jnp.float32,
                )
                p = jnp.exp(s)
                pv = lax.dot_general(
                    p.astype(jnp.bfloat16), vT, (((1,), (1,)), ((), ())),
                    preferred_element_type=jnp.float32,
                )
                lv = jnp.sum(p, axis=1, keepdims=True)
                if first:
                    acc[b, hh] = pv
                    lsum[b, hh] = jnp.broadcast_to(lv, (S, D))
                elif last:
                    a = acc[b, hh] + pv
                    l = lsum[b, hh] + jnp.broadcast_to(lv, (S, D))
                    out_ref[b, :, hh, :] = a / l
                else:
                    acc[b, hh] = acc[b, hh] + pv
                    lsum[b, hh] = lsum[b, hh] + jnp.broadcast_to(lv, (S, D))
                return 0

            lax.fori_loop(0, B * H, bh_step, 0)

        if _ABLATE == "compute":
            chunk_pass(lambda b, hh: kloc[b, hh], lambda b, hh: vloc[b, hh],
                       first=True, last=False)
            for i in range(N_DEV - 1):
                chunk_pass(lambda b, hh: kloc[b, hh],
                           lambda b, hh: vloc[b, hh],
                           first=False, last=(i == N_DEV - 2))
        else:
            if _ABLATE != "comm":
                chunk_pass(lambda b, hh: kloc[b, hh],
                           lambda b, hh: vloc[b, hh],
                           first=True, last=False)

            for i, r in enumerate(range(N_DEV - 1)):
                wk = pltpu.make_async_remote_copy(
                    src_ref=kloc, dst_ref=k_com.at[r],
                    send_sem=ksend.at[0], recv_sem=krecv.at[r],
                    device_id=(mx, my, mz),
                    device_id_type=pl.DeviceIdType.MESH,
                )
                wv = pltpu.make_async_remote_copy(
                    src_ref=vloc, dst_ref=v_com.at[r],
                    send_sem=vsend.at[0], recv_sem=vrecv.at[r],
                    device_id=(mx, my, mz),
                    device_id_type=pl.DeviceIdType.MESH,
                )
                wk.wait_recv()
                wv.wait_recv()
                if _ABLATE != "comm":
                    chunk_pass(lambda b, hh, r=r: k_com[r, b, hh],
                               lambda b, hh, r=r: v_com[r, b, hh],
                               first=False, last=(i == N_DEV - 2))
            if _ABLATE == "comm":
                out_ref[...] = q_ref[...]

        for rk, rv in sends:
            rk.wait_send()
            rv.wait_send()

    return pl.pallas_call(
        body,
        out_shape=jax.ShapeDtypeStruct((B, S, H, D), jnp.float32),
        in_specs=[pl.BlockSpec(memory_space=pltpu.VMEM)] * 3,
        out_specs=pl.BlockSpec(memory_space=pltpu.VMEM),
        scratch_shapes=[
            pltpu.VMEM((B, H, S, D), jnp.bfloat16),
            pltpu.VMEM((B, H, D, S), jnp.bfloat16),
            pltpu.VMEM((B, H, D, S), jnp.bfloat16),
            pltpu.VMEM((N_DEV - 1, B, H, D, S), jnp.bfloat16),
            pltpu.VMEM((N_DEV - 1, B, H, D, S), jnp.bfloat16),
            pltpu.VMEM((B, H, S, D), jnp.float32),
            pltpu.VMEM((B, H, S, D), jnp.float32),
            pltpu.SemaphoreType.DMA((N_DEV - 1,)),
            pltpu.SemaphoreType.DMA((N_DEV - 1,)),
            pltpu.SemaphoreType.DMA((N_DEV - 1,)),
            pltpu.SemaphoreType.DMA((N_DEV - 1,)),
        ],
        compiler_params=pltpu.CompilerParams(
            collective_id=0,
            vmem_limit_bytes=100 * 1024 * 1024,
        ),
    )(Q, K, V)


# device time: 114180 ns/iter; 1.2669x vs baseline; 1.2669x over previous
import os

import jax
import jax.numpy as jnp
from jax import lax
from jax.experimental import pallas as pl
from jax.experimental.pallas import tpu as pltpu

_ABLATE = os.environ.get("ABLATE", "")

N_DEV = 4
B, S, H, D = 2, 512, 8, 64
SCALE = D ** -0.5


def kernel(Q, K, V):
    def body(q_ref, k_ref, v_ref, out_ref,
             qt, kloc, vloc, k_com, v_com, acc, lsum,
             ksend, krecv, vsend, vrecv):
        mx = lax.axis_index("x")
        my = lax.axis_index("y")
        mz = lax.axis_index("z")

        barrier_sem = pltpu.get_barrier_semaphore()
        for j in range(N_DEV - 1):
            pl.semaphore_signal(
                barrier_sem, inc=1,
                device_id=(mx, my, (mz + 1 + j) % N_DEV),
                device_id_type=pl.DeviceIdType.MESH,
            )
        pl.semaphore_wait(barrier_sem, N_DEV - 1)

        def kv_fill(bh, _):
            b = bh // H
            hh = bh % H
            kloc[b, hh] = jnp.transpose(
                k_ref[b, :, hh, :], (1, 0)).astype(jnp.bfloat16)
            vloc[b, hh] = jnp.transpose(
                v_ref[b, :, hh, :], (1, 0)).astype(jnp.bfloat16)
            return 0

        lax.fori_loop(0, B * H, kv_fill, 0)

        sends = []
        for j in range(N_DEV - 1 if _ABLATE != "compute" else 0):
            p = (mz + 1 + j) % N_DEV
            r = N_DEV - 2 - j
            rk = pltpu.make_async_remote_copy(
                src_ref=kloc, dst_ref=k_com.at[r],
                send_sem=ksend.at[j], recv_sem=krecv.at[r],
                device_id=(mx, my, p), device_id_type=pl.DeviceIdType.MESH,
            )
            rv = pltpu.make_async_remote_copy(
                src_ref=vloc, dst_ref=v_com.at[r],
                send_sem=vsend.at[j], recv_sem=vrecv.at[r],
                device_id=(mx, my, p), device_id_type=pl.DeviceIdType.MESH,
            )
            rk.start()
            rv.start()
            sends.append((rk, rv))

        def q_fill(bh, _):
            b = bh // H
            hh = bh % H
            qt[b, hh] = (q_ref[b, :, hh, :] * SCALE).astype(jnp.bfloat16)
            return 0

        lax.fori_loop(0, B * H, q_fill, 0)

        def chunk_pass(k_at, v_at, first, last):
            def bh_step(bh, _):
                b = bh // H
                hh = bh % H
                q = qt[b, hh]
                kT = k_at(b, hh)
                vT = v_at(b, hh)
                s = lax.dot_general(
                    q, kT, (((1,), (0,)), ((), ())),
                    preferred_element_type=jnp.float32,
                )
                p = jnp.exp(s)
                pv = lax.dot_general(
                    p.astype(jnp.bfloat16), vT, (((1,), (1,)), ((), ())),
                    preferred_element_type=jnp.float32,
                )
                lv = jnp.sum(p, axis=1, keepdims=True)
                if first:
                    acc[b, hh] = pv
                    lsum[b, hh] = jnp.broadcast_to(lv, (S, D))
                elif last:
                    a = acc[b, hh] + pv
                    l = lsum[b, hh] + jnp.broadcast_to(lv, (S, D))
                    out_ref[b, :, hh, :] = a / l
                else:
                    acc[b, hh] = acc[b, hh] + pv
                    lsum[b, hh] = lsum[b, hh] + jnp.broadcast_to(lv, (S, D))
                return 0

            lax.fori_loop(0, B * H, bh_step, 0)

        if _ABLATE == "compute":
            chunk_pass(lambda b, hh: kloc[b, hh], lambda b, hh: vloc[b, hh],
                       first=True, last=False)
            for i in range(N_DEV - 1):
                chunk_pass(lambda b, hh: kloc[b, hh],
                           lambda b, hh: vloc[b, hh],
                           first=False, last=(i == N_DEV - 2))
        else:
            if _ABLATE != "comm":
                chunk_pass(lambda b, hh: kloc[b, hh],
                           lambda b, hh: vloc[b, hh],
                           first=True, last=False)

            for i, r in enumerate(range(N_DEV - 1)):
                wk = pltpu.make_async_remote_copy(
                    src_ref=kloc, dst_ref=k_com.at[r],
                    send_sem=ksend.at[0], recv_sem=krecv.at[r],
                    device_id=(mx, my, mz),
                    device_id_type=pl.DeviceIdType.MESH,
                )
                wv = pltpu.make_async_remote_copy(
                    src_ref=vloc, dst_ref=v_com.at[r],
                    send_sem=vsend.at[0], recv_sem=vrecv.at[r],
                    device_id=(mx, my, mz),
                    device_id_type=pl.DeviceIdType.MESH,
                )
                wk.wait_recv()
                wv.wait_recv()
                if _ABLATE != "comm":
                    chunk_pass(lambda b, hh, r=r: k_com[r, b, hh],
                               lambda b, hh, r=r: v_com[r, b, hh],
                               first=False, last=(i == N_DEV - 2))
            if _ABLATE == "comm":
                out_ref[...] = q_ref[...]

        for rk, rv in sends:
            rk.wait_send()
            rv.wait_send()

    return pl.pallas_call(
        body,
        out_shape=jax.ShapeDtypeStruct((B, S, H, D), jnp.float32),
        in_specs=[pl.BlockSpec(memory_space=pltpu.VMEM)] * 3,
        out_specs=pl.BlockSpec(memory_space=pltpu.VMEM),
        scratch_shapes=[
            pltpu.VMEM((B, H, S, D), jnp.bfloat16),
            pltpu.VMEM((B, H, D, S), jnp.bfloat16),
            pltpu.VMEM((B, H, D, S), jnp.bfloat16),
            pltpu.VMEM((N_DEV - 1, B, H, D, S), jnp.bfloat16),
            pltpu.VMEM((N_DEV - 1, B, H, D, S), jnp.bfloat16),
            pltpu.VMEM((B, H, S, D), jnp.float32),
            pltpu.VMEM((B, H, S, D), jnp.float32),
            pltpu.SemaphoreType.DMA((N_DEV - 1,)),
            pltpu.SemaphoreType.DMA((N_DEV - 1,)),
            pltpu.SemaphoreType.DMA((N_DEV - 1,)),
            pltpu.SemaphoreType.DMA((N_DEV - 1,)),
        ],
        compiler_params=pltpu.CompilerParams(
            collective_id=0,
            vmem_limit_bytes=100 * 1024 * 1024,
        ),
    )(Q, K, V)
